# baseline (speedup 1.0000x reference)
import functools

import jax
import jax.numpy as jnp
from jax import lax
from jax.experimental import pallas as pl
from jax.experimental.pallas import tpu as pltpu
from jax.experimental.pallas import tpu_sc as plsc

NUM_CORES = 2
NUM_SUBCORES = 16
NUM_WORKERS = NUM_CORES * NUM_SUBCORES

CHUNK = 32
LANES = 16


def _pos_embed_body(n_rows, d, x_hbm, pos_hbm, table_hbm, out_hbm,
                    idx_v, buf_v, rows_v, sem_x, sem_t):
    rows_per_w = n_rows // NUM_WORKERS
    n_chunks = rows_per_w // CHUNK
    vecs = (CHUNK * d) // LANES
    wid = lax.axis_index("s") * NUM_CORES + lax.axis_index("c")
    base = wid * rows_per_w

    def step(g, carry):
        row = base + g * CHUNK
        pltpu.sync_copy(pos_hbm.at[pl.ds(row, CHUNK)], idx_v)
        cp_x = pltpu.async_copy(x_hbm.at[pl.ds(row, CHUNK)], buf_v, sem_x)
        cp_t = pltpu.async_copy(table_hbm.at[idx_v], rows_v, sem_t)
        cp_x.wait()
        cp_t.wait()

        d_vecs = d // LANES

        def add_vec(i, carry2):
            r = i // d_vecs
            c = i % d_vecs
            sl = pl.ds(c * LANES, LANES)
            buf_v[r, sl] = buf_v[r, sl] + rows_v[r, sl]
            return carry2

        lax.fori_loop(0, vecs, add_vec, 0)
        pltpu.sync_copy(buf_v, out_hbm.at[pl.ds(row, CHUNK)])
        return carry

    lax.fori_loop(0, n_chunks, step, 0)


def kernel(x, x_pos, table):
    b, s, d = x.shape
    n_rows = b * s
    x2d = x.reshape(n_rows, d)
    pos = x_pos.reshape(n_rows).astype(jnp.int32)

    mesh = plsc.VectorSubcoreMesh(
        core_axis_name="c", subcore_axis_name="s",
        num_cores=NUM_CORES, num_subcores=NUM_SUBCORES)

    body = functools.partial(_pos_embed_body, n_rows, d)
    out2d = pl.kernel(
        body,
        out_type=jax.ShapeDtypeStruct((n_rows, d), jnp.float32),
        mesh=mesh,
        scratch_types=[
            pltpu.VMEM((CHUNK,), jnp.int32),
            pltpu.VMEM((CHUNK, d), jnp.float32),
            pltpu.VMEM((CHUNK, d), jnp.float32),
            pltpu.SemaphoreType.DMA,
            pltpu.SemaphoreType.DMA,
        ],
    )(x2d, pos, table)
    return out2d.reshape(b, s, d)


# speedup vs baseline: 2.9348x; 2.9348x over previous
import functools

import jax
import jax.numpy as jnp
from jax import lax
from jax.experimental import pallas as pl
from jax.experimental.pallas import tpu as pltpu
from jax.experimental.pallas import tpu_sc as plsc

NUM_CORES = 2
NUM_SUBCORES = 16
NUM_WORKERS = NUM_CORES * NUM_SUBCORES

CHUNK = 16
NBUF = 2
LANES = 16


def _pos_embed_body(n_rows, d, x_hbm, pos_hbm, table_hbm, out_hbm,
                    idx_v, buf_v, rows_v, sem_in, sem_out):
    rows_per_w = n_rows // NUM_WORKERS
    n_chunks = rows_per_w // CHUNK
    d_vecs = d // LANES
    wid = lax.axis_index("s") * NUM_CORES + lax.axis_index("c")
    base = wid * rows_per_w

    pltpu.sync_copy(pos_hbm.at[pl.ds(base, rows_per_w)], idx_v)

    def start_loads(g, b):
        row = pl.ds(base + g * CHUNK, CHUNK)
        idx = idx_v.at[pl.ds(g * CHUNK, CHUNK)]
        cp_x = pltpu.async_copy(x_hbm.at[row], buf_v.at[b], sem_in.at[b])
        cp_t = pltpu.async_copy(table_hbm.at[idx], rows_v.at[b], sem_in.at[b])
        return cp_x, cp_t

    def process(g, b):
        other = 1 - b

        @pl.when(g >= 1)
        def _():
            pltpu.make_async_copy(
                buf_v.at[other], out_hbm.at[pl.ds(base, CHUNK)],
                sem_out.at[other]).wait()

        @pl.when(g + 1 < n_chunks)
        def _():
            start_loads(g + 1, other)

        row = pl.ds(base + g * CHUNK, CHUNK)
        idx = idx_v.at[pl.ds(g * CHUNK, CHUNK)]
        pltpu.make_async_copy(x_hbm.at[row], buf_v.at[b], sem_in.at[b]).wait()
        pltpu.make_async_copy(table_hbm.at[idx], rows_v.at[b],
                              sem_in.at[b]).wait()

        buf = buf_v.at[b]
        rows = rows_v.at[b]

        @plsc.parallel_loop(0, CHUNK * d_vecs, unroll=8)
        def _(i):
            r = i // d_vecs
            sl = pl.ds((i % d_vecs) * LANES, LANES)
            buf[r, sl] = buf[r, sl] + rows[r, sl]

        pltpu.async_copy(buf_v.at[b], out_hbm.at[row], sem_out.at[b])

    def super_step(gg, carry):
        for b in range(NBUF):
            process(gg * NBUF + b, b)
        return carry

    start_loads(0, 0)
    lax.fori_loop(0, n_chunks // NBUF, super_step, 0)

    b_last = (n_chunks - 1) % NBUF
    pltpu.make_async_copy(
        buf_v.at[b_last], out_hbm.at[pl.ds(base, CHUNK)],
        sem_out.at[b_last]).wait()


def kernel(x, x_pos, table):
    b, s, d = x.shape
    n_rows = b * s
    x2d = x.reshape(n_rows, d)
    pos = x_pos.reshape(n_rows).astype(jnp.int32)

    mesh = plsc.VectorSubcoreMesh(
        core_axis_name="c", subcore_axis_name="s",
        num_cores=NUM_CORES, num_subcores=NUM_SUBCORES)

    rows_per_w = n_rows // NUM_WORKERS
    body = functools.partial(_pos_embed_body, n_rows, d)
    out2d = pl.kernel(
        body,
        out_type=jax.ShapeDtypeStruct((n_rows, d), jnp.float32),
        mesh=mesh,
        scratch_types=[
            pltpu.VMEM((rows_per_w,), jnp.int32),
            pltpu.VMEM((NBUF, CHUNK, d), jnp.float32),
            pltpu.VMEM((NBUF, CHUNK, d), jnp.float32),
            pltpu.SemaphoreType.DMA((NBUF,)),
            pltpu.SemaphoreType.DMA((NBUF,)),
        ],
    )(x2d, pos, table)
    return out2d.reshape(b, s, d)


# speedup vs baseline: 3.0512x; 1.0397x over previous
import functools

import jax
import jax.numpy as jnp
from jax import lax
from jax.experimental import pallas as pl
from jax.experimental.pallas import tpu as pltpu
from jax.experimental.pallas import tpu_sc as plsc

NUM_CORES = 2
NUM_SUBCORES = 16
NUM_WORKERS = NUM_CORES * NUM_SUBCORES

CHUNK = 8
NBUF = 4
LANES = 16


def _pos_embed_body(n_rows, d, x_hbm, pos_hbm, table_hbm, out_hbm,
                    idx_v, buf_v, rows_v, sem_in, sem_out):
    rows_per_w = n_rows // NUM_WORKERS
    n_chunks = rows_per_w // CHUNK
    d_vecs = d // LANES
    wid = lax.axis_index("s") * NUM_CORES + lax.axis_index("c")
    base = wid * rows_per_w

    pltpu.sync_copy(pos_hbm.at[pl.ds(base, rows_per_w)], idx_v)

    def start_loads(g, b):
        row = pl.ds(base + g * CHUNK, CHUNK)
        idx = idx_v.at[pl.ds(g * CHUNK, CHUNK)]
        cp_x = pltpu.async_copy(x_hbm.at[row], buf_v.at[b], sem_in.at[b])
        cp_t = pltpu.async_copy(table_hbm.at[idx], rows_v.at[b], sem_in.at[b])
        return cp_x, cp_t

    PREFETCH = NBUF - 2

    def process(g, b):
        nxt = g + PREFETCH
        target = (b + PREFETCH) % NBUF

        @pl.when(nxt >= NBUF)
        def _():
            pltpu.make_async_copy(
                buf_v.at[target], out_hbm.at[pl.ds(base, CHUNK)],
                sem_out.at[target]).wait()

        @pl.when(nxt < n_chunks)
        def _():
            start_loads(nxt, target)

        row = pl.ds(base + g * CHUNK, CHUNK)
        idx = idx_v.at[pl.ds(g * CHUNK, CHUNK)]
        pltpu.make_async_copy(x_hbm.at[row], buf_v.at[b], sem_in.at[b]).wait()
        pltpu.make_async_copy(table_hbm.at[idx], rows_v.at[b],
                              sem_in.at[b]).wait()

        buf = buf_v.at[b]
        rows = rows_v.at[b]

        @plsc.parallel_loop(0, CHUNK * d_vecs, unroll=8)
        def _(i):
            r = i // d_vecs
            sl = pl.ds((i % d_vecs) * LANES, LANES)
            buf[r, sl] = buf[r, sl] + rows[r, sl]

        pltpu.async_copy(buf_v.at[b], out_hbm.at[row], sem_out.at[b])

    def super_step(gg, carry):
        for b in range(NBUF):
            process(gg * NBUF + b, b)
        return carry

    for g in range(PREFETCH):
        start_loads(g, g % NBUF)
    lax.fori_loop(0, n_chunks // NBUF, super_step, 0)

    for g in range(n_chunks - PREFETCH, n_chunks):
        b_last = g % NBUF
        pltpu.make_async_copy(
            buf_v.at[b_last], out_hbm.at[pl.ds(base, CHUNK)],
            sem_out.at[b_last]).wait()


def kernel(x, x_pos, table):
    b, s, d = x.shape
    n_rows = b * s
    x2d = x.reshape(n_rows, d)
    pos = x_pos.reshape(n_rows).astype(jnp.int32)

    mesh = plsc.VectorSubcoreMesh(
        core_axis_name="c", subcore_axis_name="s",
        num_cores=NUM_CORES, num_subcores=NUM_SUBCORES)

    rows_per_w = n_rows // NUM_WORKERS
    body = functools.partial(_pos_embed_body, n_rows, d)
    out2d = pl.kernel(
        body,
        out_type=jax.ShapeDtypeStruct((n_rows, d), jnp.float32),
        mesh=mesh,
        scratch_types=[
            pltpu.VMEM((rows_per_w,), jnp.int32),
            pltpu.VMEM((NBUF, CHUNK, d), jnp.float32),
            pltpu.VMEM((NBUF, CHUNK, d), jnp.float32),
            pltpu.SemaphoreType.DMA((NBUF,)),
            pltpu.SemaphoreType.DMA((NBUF,)),
        ],
    )(x2d, pos, table)
    return out2d.reshape(b, s, d)
